# baseline (speedup 1.0000x reference)
import functools

import jax
import jax.numpy as jnp
from jax import lax
from jax.experimental import pallas as pl
from jax.experimental.pallas import tpu as pltpu
from jax.experimental.pallas import tpu_sc as plsc

NC = 2
NS = 16
NW = NC * NS
LANES = 16
NPAD = 10240
BM = 1024
K = 128
NCHT = 80
EPT = NCHT * K
EPAD = EPT * NW
NHIST = 8

_MESH = dict(core_axis_name="c", subcore_axis_name="s", num_cores=NC,
             num_subcores=NS)


def _sc_degree(dstp):
    @functools.partial(
        pl.kernel,
        out_type=jax.ShapeDtypeStruct((NW, NPAD), jnp.float32),
        mesh=plsc.VectorSubcoreMesh(**_MESH),
        scratch_types=[
            pltpu.VMEM((EPT + LANES,), jnp.int32),
            pltpu.VMEM((NPAD,), jnp.float32),
        ],
    )
    def k(dst_hbm, out_hbm, idx_v, hist_v):
        wid = lax.axis_index("s") * NC + lax.axis_index("c")
        pltpu.sync_copy(dst_hbm.at[pl.ds(wid * EPT, EPT)],
                        idx_v.at[pl.ds(0, EPT)])

        z16 = jnp.zeros((LANES,), jnp.float32)

        def zero(i, carry):
            hist_v[pl.ds(i * LANES, LANES)] = z16
            return carry

        lax.fori_loop(0, NPAD // LANES, zero, 0)

        onehot = jnp.where(lax.iota(jnp.int32, LANES) == 0,
                           jnp.float32(1.0), jnp.float32(0.0))

        def body(e, carry):
            i = idx_v[pl.ds(e, LANES)][0]
            hist_v[pl.ds(i, LANES)] = hist_v[pl.ds(i, LANES)] + onehot
            return carry

        lax.fori_loop(0, EPT, body, 0)
        pltpu.sync_copy(hist_v.at[pl.ds(0, NPAD)], out_hbm.at[wid])

    return k(dstp)


def _sc_conv(g, srcf, dstf):
    H = g.shape[1]
    rows_per_tile = NPAD // NS

    @functools.partial(
        pl.kernel,
        out_type=jax.ShapeDtypeStruct((NC, NPAD, H), jnp.float32),
        mesh=plsc.VectorSubcoreMesh(**_MESH),
        scratch_types=[
            pltpu.VMEM((K,), jnp.int32),
            pltpu.VMEM((K,), jnp.int32),
            pltpu.VMEM((K, H), jnp.float32),
            pltpu.VMEM_SHARED((NPAD, H), jnp.float32),
            pltpu.SemaphoreType.DMA,
        ],
    )
    def k(g_hbm, src_hbm, dst_hbm, out_hbm, di0, di1, rows0,
          acc_sh, sg0):
        cid = lax.axis_index("c")
        sid = lax.axis_index("s")
        wid = sid * NC + cid
        z16 = jnp.zeros((LANES,), jnp.float32)

        def zrow(r, carry):
            for c in range(H // LANES):
                rows0[r, pl.ds(c * LANES, LANES)] = z16
            return carry

        lax.fori_loop(0, K, zrow, 0)
        for i in range(rows_per_tile // K):
            pltpu.sync_copy(
                rows0, acc_sh.at[pl.ds(sid * rows_per_tile + i * K, K)])
        plsc.subcore_barrier()

        def chunk(j, carry):
            base = wid * EPT + j * K
            pltpu.sync_copy(src_hbm.at[pl.ds(base, K)], di1)
            pltpu.sync_copy(dst_hbm.at[pl.ds(base, K)], di0)
            pltpu.async_copy(g_hbm.at[di1], rows0, sg0).wait()
            pltpu.sync_copy(rows0, acc_sh.at[di0], add=True)
            return carry

        lax.fori_loop(0, NCHT, chunk, 0)
        plsc.subcore_barrier()
        pltpu.sync_copy(
            acc_sh.at[pl.ds(sid * rows_per_tile, rows_per_tile)],
            out_hbm.at[cid, pl.ds(sid * rows_per_tile, rows_per_tile)])

    return k(g, srcf, dstf)


def _sc_predict(A, B, row2, col2, ea2, de, wc, bp1, wp2, bp2pad):
    H = A.shape[1]
    DE = de
    HC = H // LANES
    KD = K * DE

    @functools.partial(
        pl.kernel,
        out_type=jax.ShapeDtypeStruct((EPAD,), jnp.float32),
        mesh=plsc.VectorSubcoreMesh(**_MESH),
        scratch_types=[
            pltpu.VMEM((NCHT, K), jnp.int32),
            pltpu.VMEM((NCHT, K), jnp.int32),
            pltpu.VMEM((K, H), jnp.float32),
            pltpu.VMEM((K, H), jnp.float32),
            pltpu.VMEM((K, H), jnp.float32),
            pltpu.VMEM((K, H), jnp.float32),
            pltpu.VMEM((KD + LANES,), jnp.float32),
            pltpu.VMEM((KD + LANES,), jnp.float32),
            pltpu.VMEM((EPT,), jnp.float32),
            pltpu.VMEM((DE, H), jnp.float32),
            pltpu.VMEM((H,), jnp.float32),
            pltpu.VMEM((H,), jnp.float32),
            pltpu.VMEM((LANES,), jnp.float32),
            pltpu.SemaphoreType.DMA,
            pltpu.SemaphoreType.DMA,
            pltpu.SemaphoreType.DMA,
            pltpu.SemaphoreType.DMA,
            pltpu.SemaphoreType.DMA,
            pltpu.SemaphoreType.DMA,
        ],
    )
    def k(a_hbm, b_hbm, row_hbm, col_hbm, ea_hbm, wc_hbm, bp1_hbm,
          wp2_hbm, bp2_hbm, out_hbm, ri_v, ci_v, ar0, ar1, br0, br1,
          ea0, ea1, out_v, wc_v, bp1_v, wp2_v, bp2_v,
          sa0, sb0, se0, sa1, sb1, se1):
        wid = lax.axis_index("s") * NC + lax.axis_index("c")
        pltpu.sync_copy(wc_hbm, wc_v)
        pltpu.sync_copy(bp1_hbm, bp1_v)
        pltpu.sync_copy(wp2_hbm, wp2_v)
        pltpu.sync_copy(bp2_hbm, bp2_v)
        pltpu.sync_copy(row_hbm.at[pl.ds(wid * NCHT, NCHT)], ri_v)
        pltpu.sync_copy(col_hbm.at[pl.ds(wid * NCHT, NCHT)], ci_v)
        b2v = bp2_v[pl.ds(0, LANES)]
        b2s16 = jnp.full((LANES,), b2v[0] * jnp.float32(1.0 / LANES),
                         jnp.float32)
        lane = lax.iota(jnp.int32, LANES)
        xor_idx = [jnp.bitwise_xor(lane, kk) for kk in (1, 2, 4, 8)]
        bp1c = [bp1_v[pl.ds(LANES * c, LANES)] for c in range(HC)]
        wp2c = [wp2_v[pl.ds(LANES * c, LANES)] for c in range(HC)]
        wcc = [[wc_v[j, pl.ds(LANES * c, LANES)] for c in range(HC)]
               for j in range(DE)]

        def issue(j, ar_v, br_v, ea_v, sa, sb, se):
            pltpu.async_copy(a_hbm.at[ri_v.at[j]], ar_v, sa)
            pltpu.async_copy(b_hbm.at[ci_v.at[j]], br_v, sb)
            pltpu.async_copy(ea_hbm.at[wid * NCHT + j],
                             ea_v.at[pl.ds(0, KD)], se)

        def waitbuf(j, ar_v, br_v, ea_v, sa, sb, se):
            pltpu.make_async_copy(a_hbm.at[ri_v.at[j]], ar_v, sa).wait()
            pltpu.make_async_copy(b_hbm.at[ci_v.at[j]], br_v, sb).wait()
            pltpu.make_async_copy(ea_hbm.at[wid * NCHT + j],
                                  ea_v.at[pl.ds(0, KD)], se).wait()

        def compute(j, ar_v, br_v, ea_v):
            def group(gi, c2):
                res = jnp.zeros((LANES,), jnp.float32)
                for el in range(LANES):
                    e = gi * LANES + el
                    ev = ea_v[pl.ds(e * DE, LANES)]
                    dacc = b2s16
                    for c in range(HC):
                        acc = bp1c[c]
                        for d in range(DE):
                            acc = acc + ev[d] * wcc[d][c]
                        acc = acc + ar_v[e, pl.ds(LANES * c, LANES)]
                        acc = acc + br_v[e, pl.ds(LANES * c, LANES)]
                        acc = jnp.maximum(acc, 0.0)
                        dacc = dacc + acc * wp2c[c]
                    for xi in xor_idx:
                        dacc = dacc + dacc.at[xi].get(
                            mode="promise_in_bounds", unique_indices=True)
                    res = jnp.where(lane == el, dacc, res)
                out_v[pl.ds(j * K + gi * LANES, LANES)] = res
                return c2

            lax.fori_loop(0, K // LANES, group, 0)

        issue(0, ar0, br0, ea0, sa0, sb0, se0)

        def pair(p, carry):
            j0 = 2 * p
            j1 = j0 + 1
            waitbuf(j0, ar0, br0, ea0, sa0, sb0, se0)
            issue(j1, ar1, br1, ea1, sa1, sb1, se1)
            compute(j0, ar0, br0, ea0)
            waitbuf(j1, ar1, br1, ea1, sa1, sb1, se1)

            @pl.when(p + 1 < NCHT // 2)
            def _():
                issue(j0 + 2, ar0, br0, ea0, sa0, sb0, se0)

            compute(j1, ar1, br1, ea1)
            return carry

        lax.fori_loop(0, NCHT // 2, pair, 0)
        pltpu.sync_copy(out_v, out_hbm.at[pl.ds(wid * EPT, EPT)])

    return k(A, B, row2, col2, ea2, wc, bp1, wp2, bp2pad)


def _tc_embed(x_pad, We, be, W1, hists):
    H = We.shape[1]

    def body(x_ref, we_ref, be_ref, w1_ref, h_ref, g_ref, dis_ref):
        h0 = jnp.maximum(
            jnp.dot(x_ref[...], we_ref[...],
                    preferred_element_type=jnp.float32) + be_ref[...], 0.0)
        deg = h_ref[0]
        for i in range(1, NW):
            deg = deg + h_ref[i]
        dis = lax.rsqrt(deg + 1.0)
        g_ref[...] = jnp.dot(h0, w1_ref[...],
                             preferred_element_type=jnp.float32) * dis
        dis_ref[...] = dis

    return pl.pallas_call(
        body,
        grid=(NPAD // BM,),
        in_specs=[
            pl.BlockSpec((BM, x_pad.shape[1]), lambda i: (i, 0)),
            pl.BlockSpec(We.shape, lambda i: (0, 0)),
            pl.BlockSpec((1, H), lambda i: (0, 0)),
            pl.BlockSpec(W1.shape, lambda i: (0, 0)),
            pl.BlockSpec((NW, BM, 1), lambda i: (0, i, 0)),
        ],
        out_specs=[
            pl.BlockSpec((BM, H), lambda i: (i, 0)),
            pl.BlockSpec((BM, 1), lambda i: (i, 0)),
        ],
        out_shape=[
            jax.ShapeDtypeStruct((NPAD, H), jnp.float32),
            jax.ShapeDtypeStruct((NPAD, 1), jnp.float32),
        ],
    )(x_pad, We, be, W1, hists)


def _tc_conv_combine(S, g, dis, b, W):
    H = g.shape[1]

    def body(s_ref, g_ref, dis_ref, b_ref, w_ref, out_ref):
        t = s_ref[0] + s_ref[1] + g_ref[...]
        h = jnp.maximum(dis_ref[...] * t + b_ref[...], 0.0)
        out_ref[...] = jnp.dot(
            h, w_ref[...], preferred_element_type=jnp.float32) * dis_ref[...]

    return pl.pallas_call(
        body,
        grid=(NPAD // BM,),
        in_specs=[
            pl.BlockSpec((NC, BM, H), lambda i: (0, i, 0)),
            pl.BlockSpec((BM, H), lambda i: (i, 0)),
            pl.BlockSpec((BM, 1), lambda i: (i, 0)),
            pl.BlockSpec((1, H), lambda i: (0, 0)),
            pl.BlockSpec((H, H), lambda i: (0, 0)),
        ],
        out_specs=pl.BlockSpec((BM, H), lambda i: (i, 0)),
        out_shape=jax.ShapeDtypeStruct((NPAD, H), jnp.float32),
    )(S, g, dis, b, W)


def _tc_final_tables(S, g, dis, b, Wa, Wb):
    H = g.shape[1]

    def body(s_ref, g_ref, dis_ref, b_ref, wa_ref, wb_ref, a_ref, bt_ref):
        t = s_ref[0] + s_ref[1] + g_ref[...]
        h = jnp.maximum(dis_ref[...] * t + b_ref[...], 0.0)
        a_ref[...] = jnp.dot(h, wa_ref[...],
                             preferred_element_type=jnp.float32)
        bt_ref[...] = jnp.dot(h, wb_ref[...],
                              preferred_element_type=jnp.float32)

    return pl.pallas_call(
        body,
        grid=(NPAD // BM,),
        in_specs=[
            pl.BlockSpec((NC, BM, H), lambda i: (0, i, 0)),
            pl.BlockSpec((BM, H), lambda i: (i, 0)),
            pl.BlockSpec((BM, 1), lambda i: (i, 0)),
            pl.BlockSpec((1, H), lambda i: (0, 0)),
            pl.BlockSpec((H, H), lambda i: (0, 0)),
            pl.BlockSpec((H, H), lambda i: (0, 0)),
        ],
        out_specs=[
            pl.BlockSpec((BM, H), lambda i: (i, 0)),
            pl.BlockSpec((BM, H), lambda i: (i, 0)),
        ],
        out_shape=[
            jax.ShapeDtypeStruct((NPAD, H), jnp.float32),
            jax.ShapeDtypeStruct((NPAD, H), jnp.float32),
        ],
    )(S, g, dis, b, Wa, Wb)


def kernel(x, edge_index, edge_attr, W_embed, b_embed, W1, b1, W2, b2,
           Wp1, bp1, Wp2, bp2):
    N, D = x.shape
    H = W1.shape[0]
    E = edge_index.shape[1]
    DE = edge_attr.shape[1]
    pad_e = EPAD - E
    padv = jnp.full((1, pad_e), NPAD - 1, jnp.int32)
    ei = jnp.concatenate(
        [edge_index, jnp.broadcast_to(padv, (2, pad_e))], axis=1)
    src2 = ei[0].reshape(EPAD // K, K)
    dst2 = ei[1].reshape(EPAD // K, K)
    ea2 = jnp.pad(edge_attr, ((0, pad_e), (0, 0))).reshape(
        EPAD // K, K * DE)

    x_pad = jnp.pad(x, ((0, NPAD - N), (0, 0)))

    hists = _sc_degree(ei[1])
    g1, dis = _tc_embed(x_pad, W_embed, b_embed[None], W1,
                        hists[..., None])
    S1 = _sc_conv(g1, ei[0], ei[1])
    g2 = _tc_conv_combine(S1, g1, dis, b1[None], W2)
    S2 = _sc_conv(g2, ei[0], ei[1])
    A, B = _tc_final_tables(S2, g2, dis, b2[None], Wp1[:H], Wp1[H:2 * H])
    outp = _sc_predict(A, B, src2, dst2, ea2, DE, Wp1[2 * H:], bp1,
                       Wp2[:, 0], jnp.pad(bp2, (0, LANES - 1)))
    return outp[:E]


# speedup vs baseline: 1.8023x; 1.8023x over previous
import functools

import jax
import jax.numpy as jnp
from jax import lax
from jax.experimental import pallas as pl
from jax.experimental.pallas import tpu as pltpu
from jax.experimental.pallas import tpu_sc as plsc

NC = 2
NS = 16
NW = NC * NS
LANES = 16
NPAD = 10240
BM = 1024
K = 128
NCHT = 80
EPT = NCHT * K
EPAD = EPT * NW
NHIST = 8

_MESH = dict(core_axis_name="c", subcore_axis_name="s", num_cores=NC,
             num_subcores=NS)


def _sc_degree(dstp):
    @functools.partial(
        pl.kernel,
        out_type=jax.ShapeDtypeStruct((NW, NPAD), jnp.float32),
        mesh=plsc.VectorSubcoreMesh(**_MESH),
        scratch_types=[
            pltpu.VMEM((EPT + LANES,), jnp.int32),
            pltpu.VMEM((NPAD,), jnp.float32),
        ],
    )
    def k(dst_hbm, out_hbm, idx_v, hist_v):
        wid = lax.axis_index("s") * NC + lax.axis_index("c")
        pltpu.sync_copy(dst_hbm.at[pl.ds(wid * EPT, EPT)],
                        idx_v.at[pl.ds(0, EPT)])

        z16 = jnp.zeros((LANES,), jnp.float32)

        def zero(i, carry):
            hist_v[pl.ds(i * LANES, LANES)] = z16
            return carry

        lax.fori_loop(0, NPAD // LANES, zero, 0)

        onehot = jnp.where(lax.iota(jnp.int32, LANES) == 0,
                           jnp.float32(1.0), jnp.float32(0.0))

        def body(e, carry):
            i = idx_v[pl.ds(e, LANES)][0]
            hist_v[pl.ds(i, LANES)] = hist_v[pl.ds(i, LANES)] + onehot
            return carry

        lax.fori_loop(0, EPT, body, 0)
        pltpu.sync_copy(hist_v.at[pl.ds(0, NPAD)], out_hbm.at[wid])

    return k(dstp)


def _sc_conv(g, srcf, dstf):
    H = g.shape[1]
    rows_per_tile = NPAD // NS

    @functools.partial(
        pl.kernel,
        out_type=jax.ShapeDtypeStruct((NC, NPAD, H), jnp.float32),
        mesh=plsc.VectorSubcoreMesh(**_MESH),
        scratch_types=[
            pltpu.VMEM((K,), jnp.int32),
            pltpu.VMEM((K,), jnp.int32),
            pltpu.VMEM((K, H), jnp.float32),
            pltpu.VMEM_SHARED((NPAD, H), jnp.float32),
            pltpu.SemaphoreType.DMA,
        ],
    )
    def k(g_hbm, src_hbm, dst_hbm, out_hbm, di0, di1, rows0,
          acc_sh, sg0):
        cid = lax.axis_index("c")
        sid = lax.axis_index("s")
        wid = sid * NC + cid
        z16 = jnp.zeros((LANES,), jnp.float32)

        def zrow(r, carry):
            for c in range(H // LANES):
                rows0[r, pl.ds(c * LANES, LANES)] = z16
            return carry

        lax.fori_loop(0, K, zrow, 0)
        for i in range(rows_per_tile // K):
            pltpu.sync_copy(
                rows0, acc_sh.at[pl.ds(sid * rows_per_tile + i * K, K)])
        plsc.subcore_barrier()

        def chunk(j, carry):
            base = wid * EPT + j * K
            pltpu.sync_copy(src_hbm.at[pl.ds(base, K)], di1)
            pltpu.sync_copy(dst_hbm.at[pl.ds(base, K)], di0)
            pltpu.async_copy(g_hbm.at[di1], rows0, sg0).wait()
            pltpu.sync_copy(rows0, acc_sh.at[di0], add=True)
            return carry

        lax.fori_loop(0, NCHT, chunk, 0)
        plsc.subcore_barrier()
        pltpu.sync_copy(
            acc_sh.at[pl.ds(sid * rows_per_tile, rows_per_tile)],
            out_hbm.at[cid, pl.ds(sid * rows_per_tile, rows_per_tile)])

    return k(g, srcf, dstf)


def _sc_predict(A, B, row2, col2, ea2, de, wc, bp1, wp2, bp2pad):
    H = A.shape[1]
    DE = de
    HC = H // LANES
    KD = K * DE

    @functools.partial(
        pl.kernel,
        out_type=jax.ShapeDtypeStruct((EPAD,), jnp.float32),
        mesh=plsc.VectorSubcoreMesh(**_MESH),
        scratch_types=[
            pltpu.VMEM((NCHT, K), jnp.int32),
            pltpu.VMEM((NCHT, K), jnp.int32),
            pltpu.VMEM((K, H), jnp.float32),
            pltpu.VMEM((K, H), jnp.float32),
            pltpu.VMEM((K, H), jnp.float32),
            pltpu.VMEM((K, H), jnp.float32),
            pltpu.VMEM((KD + LANES,), jnp.float32),
            pltpu.VMEM((KD + LANES,), jnp.float32),
            pltpu.VMEM((EPT,), jnp.float32),
            pltpu.VMEM((DE, H), jnp.float32),
            pltpu.VMEM((H,), jnp.float32),
            pltpu.VMEM((H,), jnp.float32),
            pltpu.VMEM((LANES,), jnp.float32),
            pltpu.SemaphoreType.DMA,
            pltpu.SemaphoreType.DMA,
            pltpu.SemaphoreType.DMA,
            pltpu.SemaphoreType.DMA,
            pltpu.SemaphoreType.DMA,
            pltpu.SemaphoreType.DMA,
        ],
    )
    def k(a_hbm, b_hbm, row_hbm, col_hbm, ea_hbm, wc_hbm, bp1_hbm,
          wp2_hbm, bp2_hbm, out_hbm, ri_v, ci_v, ar0, ar1, br0, br1,
          ea0, ea1, out_v, wc_v, bp1_v, wp2_v, bp2_v,
          sa0, sb0, se0, sa1, sb1, se1):
        wid = lax.axis_index("s") * NC + lax.axis_index("c")
        pltpu.sync_copy(wc_hbm, wc_v)
        pltpu.sync_copy(bp1_hbm, bp1_v)
        pltpu.sync_copy(wp2_hbm, wp2_v)
        pltpu.sync_copy(bp2_hbm, bp2_v)
        pltpu.sync_copy(row_hbm.at[pl.ds(wid * NCHT, NCHT)], ri_v)
        pltpu.sync_copy(col_hbm.at[pl.ds(wid * NCHT, NCHT)], ci_v)
        b2v = bp2_v[pl.ds(0, LANES)]
        b2s16 = jnp.full((LANES,), b2v[0] * jnp.float32(1.0 / LANES),
                         jnp.float32)
        lane = lax.iota(jnp.int32, LANES)
        xor_idx = [jnp.bitwise_xor(lane, kk) for kk in (1, 2, 4, 8)]
        bp1c = [bp1_v[pl.ds(LANES * c, LANES)] for c in range(HC)]
        wp2c = [wp2_v[pl.ds(LANES * c, LANES)] for c in range(HC)]
        wcc = [[wc_v[j, pl.ds(LANES * c, LANES)] for c in range(HC)]
               for j in range(DE)]

        def issue(j, ar_v, br_v, ea_v, sa, sb, se):
            pltpu.async_copy(a_hbm.at[ri_v.at[j]], ar_v, sa)
            pltpu.async_copy(b_hbm.at[ci_v.at[j]], br_v, sb)
            pltpu.async_copy(ea_hbm.at[wid * NCHT + j],
                             ea_v.at[pl.ds(0, KD)], se)

        def waitbuf(j, ar_v, br_v, ea_v, sa, sb, se):
            pltpu.make_async_copy(a_hbm.at[ri_v.at[j]], ar_v, sa).wait()
            pltpu.make_async_copy(b_hbm.at[ci_v.at[j]], br_v, sb).wait()
            pltpu.make_async_copy(ea_hbm.at[wid * NCHT + j],
                                  ea_v.at[pl.ds(0, KD)], se).wait()

        def compute(j, ar_v, br_v, ea_v):
            def group(gi, c2):
                res = jnp.zeros((LANES,), jnp.float32)
                for el in range(LANES):
                    e = gi * LANES + el
                    ev = ea_v[pl.ds(e * DE, LANES)]
                    dacc = b2s16
                    for c in range(HC):
                        acc = bp1c[c]
                        for d in range(DE):
                            acc = acc + ev[d] * wcc[d][c]
                        acc = acc + ar_v[e, pl.ds(LANES * c, LANES)]
                        acc = acc + br_v[e, pl.ds(LANES * c, LANES)]
                        acc = jnp.maximum(acc, 0.0)
                        dacc = dacc + acc * wp2c[c]
                    for xi in xor_idx:
                        dacc = dacc + dacc.at[xi].get(
                            mode="promise_in_bounds", unique_indices=True)
                    res = jnp.where(lane == el, dacc, res)
                out_v[pl.ds(j * K + gi * LANES, LANES)] = res
                return c2

            lax.fori_loop(0, K // LANES, group, 0)

        issue(0, ar0, br0, ea0, sa0, sb0, se0)

        def pair(p, carry):
            j0 = 2 * p
            j1 = j0 + 1
            waitbuf(j0, ar0, br0, ea0, sa0, sb0, se0)
            issue(j1, ar1, br1, ea1, sa1, sb1, se1)
            compute(j0, ar0, br0, ea0)
            waitbuf(j1, ar1, br1, ea1, sa1, sb1, se1)

            @pl.when(p + 1 < NCHT // 2)
            def _():
                issue(j0 + 2, ar0, br0, ea0, sa0, sb0, se0)

            compute(j1, ar1, br1, ea1)
            return carry

        lax.fori_loop(0, NCHT // 2, pair, 0)
        pltpu.sync_copy(out_v, out_hbm.at[pl.ds(wid * EPT, EPT)])

    return k(A, B, row2, col2, ea2, wc, bp1, wp2, bp2pad)


def _tc_embed(x_pad, We, be, W1, hists):
    H = We.shape[1]

    def body(x_ref, we_ref, be_ref, w1_ref, h_ref, g_ref, dis_ref):
        h0 = jnp.maximum(
            jnp.dot(x_ref[...], we_ref[...],
                    preferred_element_type=jnp.float32) + be_ref[...], 0.0)
        deg = h_ref[0]
        for i in range(1, NW):
            deg = deg + h_ref[i]
        dis = lax.rsqrt(deg + 1.0)
        g_ref[...] = jnp.dot(h0, w1_ref[...],
                             preferred_element_type=jnp.float32) * dis
        dis_ref[...] = dis

    return pl.pallas_call(
        body,
        grid=(NPAD // BM,),
        in_specs=[
            pl.BlockSpec((BM, x_pad.shape[1]), lambda i: (i, 0)),
            pl.BlockSpec(We.shape, lambda i: (0, 0)),
            pl.BlockSpec((1, H), lambda i: (0, 0)),
            pl.BlockSpec(W1.shape, lambda i: (0, 0)),
            pl.BlockSpec((NW, BM, 1), lambda i: (0, i, 0)),
        ],
        out_specs=[
            pl.BlockSpec((BM, H), lambda i: (i, 0)),
            pl.BlockSpec((BM, 1), lambda i: (i, 0)),
        ],
        out_shape=[
            jax.ShapeDtypeStruct((NPAD, H), jnp.float32),
            jax.ShapeDtypeStruct((NPAD, 1), jnp.float32),
        ],
    )(x_pad, We, be, W1, hists)


def _tc_conv_combine(S, g, dis, b, W):
    H = g.shape[1]

    def body(s_ref, g_ref, dis_ref, b_ref, w_ref, out_ref):
        t = s_ref[0] + s_ref[1] + g_ref[...]
        h = jnp.maximum(dis_ref[...] * t + b_ref[...], 0.0)
        out_ref[...] = jnp.dot(
            h, w_ref[...], preferred_element_type=jnp.float32) * dis_ref[...]

    return pl.pallas_call(
        body,
        grid=(NPAD // BM,),
        in_specs=[
            pl.BlockSpec((NC, BM, H), lambda i: (0, i, 0)),
            pl.BlockSpec((BM, H), lambda i: (i, 0)),
            pl.BlockSpec((BM, 1), lambda i: (i, 0)),
            pl.BlockSpec((1, H), lambda i: (0, 0)),
            pl.BlockSpec((H, H), lambda i: (0, 0)),
        ],
        out_specs=pl.BlockSpec((BM, H), lambda i: (i, 0)),
        out_shape=jax.ShapeDtypeStruct((NPAD, H), jnp.float32),
    )(S, g, dis, b, W)


def _tc_final_tables(S, g, dis, b, Wa, Wb):
    H = g.shape[1]

    def body(s_ref, g_ref, dis_ref, b_ref, wa_ref, wb_ref, a_ref, bt_ref):
        t = s_ref[0] + s_ref[1] + g_ref[...]
        h = jnp.maximum(dis_ref[...] * t + b_ref[...], 0.0)
        a_ref[...] = jnp.dot(h, wa_ref[...],
                             preferred_element_type=jnp.float32)
        bt_ref[...] = jnp.dot(h, wb_ref[...],
                              preferred_element_type=jnp.float32)

    return pl.pallas_call(
        body,
        grid=(NPAD // BM,),
        in_specs=[
            pl.BlockSpec((NC, BM, H), lambda i: (0, i, 0)),
            pl.BlockSpec((BM, H), lambda i: (i, 0)),
            pl.BlockSpec((BM, 1), lambda i: (i, 0)),
            pl.BlockSpec((1, H), lambda i: (0, 0)),
            pl.BlockSpec((H, H), lambda i: (0, 0)),
            pl.BlockSpec((H, H), lambda i: (0, 0)),
        ],
        out_specs=[
            pl.BlockSpec((BM, H), lambda i: (i, 0)),
            pl.BlockSpec((BM, H), lambda i: (i, 0)),
        ],
        out_shape=[
            jax.ShapeDtypeStruct((NPAD, H), jnp.float32),
            jax.ShapeDtypeStruct((NPAD, H), jnp.float32),
        ],
    )(S, g, dis, b, Wa, Wb)


def kernel(x, edge_index, edge_attr, W_embed, b_embed, W1, b1, W2, b2,
           Wp1, bp1, Wp2, bp2):
    N, D = x.shape
    H = W1.shape[0]
    E = edge_index.shape[1]
    DE = edge_attr.shape[1]
    pad_e = EPAD - E
    padr = N + jax.lax.rem(jnp.arange(pad_e, dtype=jnp.int32),
                           jnp.int32(NPAD - N))
    ei = jnp.concatenate(
        [edge_index, jnp.stack([padr, padr])], axis=1)
    src2 = ei[0].reshape(EPAD // K, K)
    dst2 = ei[1].reshape(EPAD // K, K)
    ea2 = jnp.pad(edge_attr, ((0, pad_e), (0, 0))).reshape(
        EPAD // K, K * DE)

    x_pad = jnp.pad(x, ((0, NPAD - N), (0, 0)))

    hists = _sc_degree(ei[1])
    g1, dis = _tc_embed(x_pad, W_embed, b_embed[None], W1,
                        hists[..., None])
    S1 = _sc_conv(g1, ei[0], ei[1])
    g2 = _tc_conv_combine(S1, g1, dis, b1[None], W2)
    S2 = _sc_conv(g2, ei[0], ei[1])
    A, B = _tc_final_tables(S2, g2, dis, b2[None], Wp1[:H], Wp1[H:2 * H])
    outp = _sc_predict(A, B, src2, dst2, ea2, DE, Wp1[2 * H:], bp1,
                       Wp2[:, 0], jnp.pad(bp2, (0, LANES - 1)))
    return outp[:E]


# speedup vs baseline: 1.9515x; 1.0828x over previous
import functools

import jax
import jax.numpy as jnp
from jax import lax
from jax.experimental import pallas as pl
from jax.experimental.pallas import tpu as pltpu
from jax.experimental.pallas import tpu_sc as plsc

NC = 2
NS = 16
NW = NC * NS
LANES = 16
NPAD = 10240
BM = 1024
K = 128
NCHT = 80
EPT = NCHT * K
EPAD = EPT * NW
NHIST = 8

_MESH = dict(core_axis_name="c", subcore_axis_name="s", num_cores=NC,
             num_subcores=NS)


def _sc_degree(dstp):
    @functools.partial(
        pl.kernel,
        out_type=jax.ShapeDtypeStruct((NW, NPAD), jnp.float32),
        mesh=plsc.VectorSubcoreMesh(**_MESH),
        scratch_types=[
            pltpu.VMEM((EPT + LANES,), jnp.int32),
            pltpu.VMEM((NPAD,), jnp.float32),
        ],
    )
    def k(dst_hbm, out_hbm, idx_v, hist_v):
        wid = lax.axis_index("s") * NC + lax.axis_index("c")
        pltpu.sync_copy(dst_hbm.at[pl.ds(wid * EPT, EPT)],
                        idx_v.at[pl.ds(0, EPT)])

        z16 = jnp.zeros((LANES,), jnp.float32)

        def zero(i, carry):
            hist_v[pl.ds(i * LANES, LANES)] = z16
            return carry

        lax.fori_loop(0, NPAD // LANES, zero, 0)

        onehot = jnp.where(lax.iota(jnp.int32, LANES) == 0,
                           jnp.float32(1.0), jnp.float32(0.0))

        def body(e, carry):
            i = idx_v[pl.ds(e, LANES)][0]
            hist_v[pl.ds(i, LANES)] = hist_v[pl.ds(i, LANES)] + onehot
            return carry

        lax.fori_loop(0, EPT, body, 0)
        pltpu.sync_copy(hist_v.at[pl.ds(0, NPAD)], out_hbm.at[wid])

    return k(dstp)


def _sc_conv(g, srcf, dstf):
    H = g.shape[1]
    rows_per_tile = NPAD // NS

    @functools.partial(
        pl.kernel,
        out_type=jax.ShapeDtypeStruct((NC, NPAD, H), jnp.float32),
        mesh=plsc.VectorSubcoreMesh(**_MESH),
        scratch_types=[
            pltpu.VMEM((K,), jnp.int32),
            pltpu.VMEM((K,), jnp.int32),
            pltpu.VMEM((K,), jnp.int32),
            pltpu.VMEM((K,), jnp.int32),
            pltpu.VMEM((K, H), jnp.float32),
            pltpu.VMEM((K, H), jnp.float32),
            pltpu.VMEM_SHARED((NPAD, H), jnp.float32),
            pltpu.SemaphoreType.DMA,
            pltpu.SemaphoreType.DMA,
            pltpu.SemaphoreType.DMA,
            pltpu.SemaphoreType.DMA,
            pltpu.SemaphoreType.DMA,
            pltpu.SemaphoreType.DMA,
        ],
    )
    def k(g_hbm, src_hbm, dst_hbm, out_hbm, si0, si1, di0, di1, rows0,
          rows1, acc_sh, sg0, sg1, ss0, ss1, sd0, sd1):
        cid = lax.axis_index("c")
        sid = lax.axis_index("s")
        wid = sid * NC + cid
        z16 = jnp.zeros((LANES,), jnp.float32)

        def zrow(r, carry):
            for c in range(H // LANES):
                rows0[r, pl.ds(c * LANES, LANES)] = z16
            return carry

        lax.fori_loop(0, K, zrow, 0)
        for i in range(rows_per_tile // K):
            pltpu.sync_copy(
                rows0, acc_sh.at[pl.ds(sid * rows_per_tile + i * K, K)])
        plsc.subcore_barrier()

        def issue_idx(j, si, di, ss, sd):
            base = wid * EPT + j * K
            pltpu.async_copy(src_hbm.at[pl.ds(base, K)], si, ss)
            pltpu.async_copy(dst_hbm.at[pl.ds(base, K)], di, sd)

        def wait_sem(hbm, buf, sem):
            pltpu.make_async_copy(hbm.at[pl.ds(0, K)], buf, sem).wait()

        issue_idx(0, si0, di0, ss0, sd0)
        wait_sem(src_hbm, si0, ss0)
        pltpu.async_copy(g_hbm.at[si0], rows0, sg0)
        issue_idx(1, si1, di1, ss1, sd1)

        def pair(p, carry):
            j0 = 2 * p
            j1 = j0 + 1
            pltpu.make_async_copy(g_hbm.at[si0], rows0, sg0).wait()
            wait_sem(dst_hbm, di0, sd0)
            wait_sem(src_hbm, si1, ss1)
            wait_sem(dst_hbm, di1, sd1)
            pltpu.async_copy(g_hbm.at[si1], rows1, sg1)
            pltpu.sync_copy(rows0, acc_sh.at[di0], add=True)

            @pl.when(p + 1 < NCHT // 2)
            def _():
                issue_idx(j0 + 2, si0, di0, ss0, sd0)
                wait_sem(src_hbm, si0, ss0)
                pltpu.async_copy(g_hbm.at[si0], rows0, sg0)

            pltpu.make_async_copy(g_hbm.at[si1], rows1, sg1).wait()
            pltpu.sync_copy(rows1, acc_sh.at[di1], add=True)

            @pl.when(p + 1 < NCHT // 2)
            def _():
                issue_idx(j1 + 2, si1, di1, ss1, sd1)

            return carry

        lax.fori_loop(0, NCHT // 2, pair, 0)
        plsc.subcore_barrier()
        pltpu.sync_copy(
            acc_sh.at[pl.ds(sid * rows_per_tile, rows_per_tile)],
            out_hbm.at[cid, pl.ds(sid * rows_per_tile, rows_per_tile)])

    return k(g, srcf, dstf)


def _sc_predict(A, B, row2, col2, ea2, de, wc, bp1, wp2, bp2pad):
    H = A.shape[1]
    DE = de
    HC = H // LANES
    KD = K * DE

    @functools.partial(
        pl.kernel,
        out_type=jax.ShapeDtypeStruct((EPAD,), jnp.float32),
        mesh=plsc.VectorSubcoreMesh(**_MESH),
        scratch_types=[
            pltpu.VMEM((NCHT, K), jnp.int32),
            pltpu.VMEM((NCHT, K), jnp.int32),
            pltpu.VMEM((K, H), jnp.float32),
            pltpu.VMEM((K, H), jnp.float32),
            pltpu.VMEM((K, H), jnp.float32),
            pltpu.VMEM((K, H), jnp.float32),
            pltpu.VMEM((KD + LANES,), jnp.float32),
            pltpu.VMEM((KD + LANES,), jnp.float32),
            pltpu.VMEM((EPT,), jnp.float32),
            pltpu.VMEM((DE, H), jnp.float32),
            pltpu.VMEM((H,), jnp.float32),
            pltpu.VMEM((H,), jnp.float32),
            pltpu.VMEM((LANES,), jnp.float32),
            pltpu.SemaphoreType.DMA,
            pltpu.SemaphoreType.DMA,
            pltpu.SemaphoreType.DMA,
            pltpu.SemaphoreType.DMA,
            pltpu.SemaphoreType.DMA,
            pltpu.SemaphoreType.DMA,
        ],
    )
    def k(a_hbm, b_hbm, row_hbm, col_hbm, ea_hbm, wc_hbm, bp1_hbm,
          wp2_hbm, bp2_hbm, out_hbm, ri_v, ci_v, ar0, ar1, br0, br1,
          ea0, ea1, out_v, wc_v, bp1_v, wp2_v, bp2_v,
          sa0, sb0, se0, sa1, sb1, se1):
        wid = lax.axis_index("s") * NC + lax.axis_index("c")
        pltpu.sync_copy(wc_hbm, wc_v)
        pltpu.sync_copy(bp1_hbm, bp1_v)
        pltpu.sync_copy(wp2_hbm, wp2_v)
        pltpu.sync_copy(bp2_hbm, bp2_v)
        pltpu.sync_copy(row_hbm.at[pl.ds(wid * NCHT, NCHT)], ri_v)
        pltpu.sync_copy(col_hbm.at[pl.ds(wid * NCHT, NCHT)], ci_v)
        b2v = bp2_v[pl.ds(0, LANES)]
        b2s16 = jnp.full((LANES,), b2v[0] * jnp.float32(1.0 / LANES),
                         jnp.float32)
        lane = lax.iota(jnp.int32, LANES)
        xor_idx = [jnp.bitwise_xor(lane, kk) for kk in (1, 2, 4, 8)]
        bp1c = [bp1_v[pl.ds(LANES * c, LANES)] for c in range(HC)]
        wp2c = [wp2_v[pl.ds(LANES * c, LANES)] for c in range(HC)]
        wcc = [[wc_v[j, pl.ds(LANES * c, LANES)] for c in range(HC)]
               for j in range(DE)]

        def issue(j, ar_v, br_v, ea_v, sa, sb, se):
            pltpu.async_copy(a_hbm.at[ri_v.at[j]], ar_v, sa)
            pltpu.async_copy(b_hbm.at[ci_v.at[j]], br_v, sb)
            pltpu.async_copy(ea_hbm.at[wid * NCHT + j],
                             ea_v.at[pl.ds(0, KD)], se)

        def waitbuf(j, ar_v, br_v, ea_v, sa, sb, se):
            pltpu.make_async_copy(a_hbm.at[ri_v.at[j]], ar_v, sa).wait()
            pltpu.make_async_copy(b_hbm.at[ci_v.at[j]], br_v, sb).wait()
            pltpu.make_async_copy(ea_hbm.at[wid * NCHT + j],
                                  ea_v.at[pl.ds(0, KD)], se).wait()

        def compute(j, ar_v, br_v, ea_v):
            def group(gi, c2):
                res = jnp.zeros((LANES,), jnp.float32)
                for el in range(LANES):
                    e = gi * LANES + el
                    ev = ea_v[pl.ds(e * DE, LANES)]
                    dacc = b2s16
                    for c in range(HC):
                        acc = bp1c[c]
                        for d in range(DE):
                            acc = acc + ev[d] * wcc[d][c]
                        acc = acc + ar_v[e, pl.ds(LANES * c, LANES)]
                        acc = acc + br_v[e, pl.ds(LANES * c, LANES)]
                        acc = jnp.maximum(acc, 0.0)
                        dacc = dacc + acc * wp2c[c]
                    for xi in xor_idx:
                        dacc = dacc + dacc.at[xi].get(
                            mode="promise_in_bounds", unique_indices=True)
                    res = jnp.where(lane == el, dacc, res)
                out_v[pl.ds(j * K + gi * LANES, LANES)] = res
                return c2

            lax.fori_loop(0, K // LANES, group, 0)

        issue(0, ar0, br0, ea0, sa0, sb0, se0)

        def pair(p, carry):
            j0 = 2 * p
            j1 = j0 + 1
            waitbuf(j0, ar0, br0, ea0, sa0, sb0, se0)
            issue(j1, ar1, br1, ea1, sa1, sb1, se1)
            compute(j0, ar0, br0, ea0)
            waitbuf(j1, ar1, br1, ea1, sa1, sb1, se1)

            @pl.when(p + 1 < NCHT // 2)
            def _():
                issue(j0 + 2, ar0, br0, ea0, sa0, sb0, se0)

            compute(j1, ar1, br1, ea1)
            return carry

        lax.fori_loop(0, NCHT // 2, pair, 0)
        pltpu.sync_copy(out_v, out_hbm.at[pl.ds(wid * EPT, EPT)])

    return k(A, B, row2, col2, ea2, wc, bp1, wp2, bp2pad)


def _tc_embed(x_pad, We, be, W1, hists):
    H = We.shape[1]

    def body(x_ref, we_ref, be_ref, w1_ref, h_ref, g_ref, dis_ref):
        h0 = jnp.maximum(
            jnp.dot(x_ref[...], we_ref[...],
                    preferred_element_type=jnp.float32) + be_ref[...], 0.0)
        deg = h_ref[0]
        for i in range(1, NW):
            deg = deg + h_ref[i]
        dis = lax.rsqrt(deg + 1.0)
        g_ref[...] = jnp.dot(h0, w1_ref[...],
                             preferred_element_type=jnp.float32) * dis
        dis_ref[...] = dis

    return pl.pallas_call(
        body,
        grid=(NPAD // BM,),
        in_specs=[
            pl.BlockSpec((BM, x_pad.shape[1]), lambda i: (i, 0)),
            pl.BlockSpec(We.shape, lambda i: (0, 0)),
            pl.BlockSpec((1, H), lambda i: (0, 0)),
            pl.BlockSpec(W1.shape, lambda i: (0, 0)),
            pl.BlockSpec((NW, BM, 1), lambda i: (0, i, 0)),
        ],
        out_specs=[
            pl.BlockSpec((BM, H), lambda i: (i, 0)),
            pl.BlockSpec((BM, 1), lambda i: (i, 0)),
        ],
        out_shape=[
            jax.ShapeDtypeStruct((NPAD, H), jnp.float32),
            jax.ShapeDtypeStruct((NPAD, 1), jnp.float32),
        ],
    )(x_pad, We, be, W1, hists)


def _tc_conv_combine(S, g, dis, b, W):
    H = g.shape[1]

    def body(s_ref, g_ref, dis_ref, b_ref, w_ref, out_ref):
        t = s_ref[0] + s_ref[1] + g_ref[...]
        h = jnp.maximum(dis_ref[...] * t + b_ref[...], 0.0)
        out_ref[...] = jnp.dot(
            h, w_ref[...], preferred_element_type=jnp.float32) * dis_ref[...]

    return pl.pallas_call(
        body,
        grid=(NPAD // BM,),
        in_specs=[
            pl.BlockSpec((NC, BM, H), lambda i: (0, i, 0)),
            pl.BlockSpec((BM, H), lambda i: (i, 0)),
            pl.BlockSpec((BM, 1), lambda i: (i, 0)),
            pl.BlockSpec((1, H), lambda i: (0, 0)),
            pl.BlockSpec((H, H), lambda i: (0, 0)),
        ],
        out_specs=pl.BlockSpec((BM, H), lambda i: (i, 0)),
        out_shape=jax.ShapeDtypeStruct((NPAD, H), jnp.float32),
    )(S, g, dis, b, W)


def _tc_final_tables(S, g, dis, b, Wa, Wb):
    H = g.shape[1]

    def body(s_ref, g_ref, dis_ref, b_ref, wa_ref, wb_ref, a_ref, bt_ref):
        t = s_ref[0] + s_ref[1] + g_ref[...]
        h = jnp.maximum(dis_ref[...] * t + b_ref[...], 0.0)
        a_ref[...] = jnp.dot(h, wa_ref[...],
                             preferred_element_type=jnp.float32)
        bt_ref[...] = jnp.dot(h, wb_ref[...],
                              preferred_element_type=jnp.float32)

    return pl.pallas_call(
        body,
        grid=(NPAD // BM,),
        in_specs=[
            pl.BlockSpec((NC, BM, H), lambda i: (0, i, 0)),
            pl.BlockSpec((BM, H), lambda i: (i, 0)),
            pl.BlockSpec((BM, 1), lambda i: (i, 0)),
            pl.BlockSpec((1, H), lambda i: (0, 0)),
            pl.BlockSpec((H, H), lambda i: (0, 0)),
            pl.BlockSpec((H, H), lambda i: (0, 0)),
        ],
        out_specs=[
            pl.BlockSpec((BM, H), lambda i: (i, 0)),
            pl.BlockSpec((BM, H), lambda i: (i, 0)),
        ],
        out_shape=[
            jax.ShapeDtypeStruct((NPAD, H), jnp.float32),
            jax.ShapeDtypeStruct((NPAD, H), jnp.float32),
        ],
    )(S, g, dis, b, Wa, Wb)


def kernel(x, edge_index, edge_attr, W_embed, b_embed, W1, b1, W2, b2,
           Wp1, bp1, Wp2, bp2):
    N, D = x.shape
    H = W1.shape[0]
    E = edge_index.shape[1]
    DE = edge_attr.shape[1]
    pad_e = EPAD - E
    padr = N + jax.lax.rem(jnp.arange(pad_e, dtype=jnp.int32),
                           jnp.int32(NPAD - N))
    ei = jnp.concatenate(
        [edge_index, jnp.stack([padr, padr])], axis=1)
    src2 = ei[0].reshape(EPAD // K, K)
    dst2 = ei[1].reshape(EPAD // K, K)
    ea2 = jnp.pad(edge_attr, ((0, pad_e), (0, 0))).reshape(
        EPAD // K, K * DE)

    x_pad = jnp.pad(x, ((0, NPAD - N), (0, 0)))

    hists = _sc_degree(ei[1])
    g1, dis = _tc_embed(x_pad, W_embed, b_embed[None], W1,
                        hists[..., None])
    S1 = _sc_conv(g1, ei[0], ei[1])
    g2 = _tc_conv_combine(S1, g1, dis, b1[None], W2)
    S2 = _sc_conv(g2, ei[0], ei[1])
    A, B = _tc_final_tables(S2, g2, dis, b2[None], Wp1[:H], Wp1[H:2 * H])
    outp = _sc_predict(A, B, src2, dst2, ea2, DE, Wp1[2 * H:], bp1,
                       Wp2[:, 0], jnp.pad(bp2, (0, LANES - 1)))
    return outp[:E]
